# baseline (speedup 1.0000x reference)
import functools

import jax
import jax.numpy as jnp
from jax import lax
from jax.experimental import pallas as pl
from jax.experimental.pallas import tpu as pltpu
from jax.experimental.pallas import tpu_sc as plsc

N = 10000
E = 320000
D_IN = 128
D = 256
NP = 10240
RPT = NP // 16
CHUNK = 128
EPAD = 327680

_mesh = plsc.VectorSubcoreMesh(core_axis_name="c", subcore_axis_name="s")


def _serial_edge_loop(table_hbm, ec_hbm, erow0, nchunk,
                      eb, rows_v, acc, si, gsem, extra=None):
    def ipf(t, q):
        pltpu.async_copy(ec_hbm.at[pl.ds(erow0 + 2 * t, 2)], eb[q], si[q])

    def ipw(q):
        pltpu.make_async_copy(ec_hbm.at[pl.ds(0, 2)], eb[q], si[q]).wait()

    ipf(0, 0)

    def body(t2, _):
        for u in range(2):
            t = 2 * t2 + u
            q = u
            ipw(q)
            ipf(t + 1, 1 - q)
            pltpu.async_copy(table_hbm.at[eb[q].at[0]], rows_v,
                             gsem).wait()
            pltpu.sync_copy(rows_v, acc.at[eb[q].at[1]], add=True)
            if extra is not None:
                extra(q)
        return 0

    lax.fori_loop(0, nchunk // 2, body, 0)
    ipw(0)


def _agg0_body(x_hbm, ec_hbm, z2d_hbm, z1d_hbm,
               agg_hbm, cnt_hbm,
               eb0, eb1, rows_v, ones_v, acc, cacc, si0, si1, gsem):
    c = lax.axis_index("c")
    s = lax.axis_index("s")
    w = c * 16 + s
    nchunk = EPAD // 32 // CHUNK

    r0 = s * RPT
    pltpu.sync_copy(z2d_hbm.at[pl.ds(r0, RPT)], acc.at[pl.ds(r0, RPT)])
    pltpu.sync_copy(z1d_hbm.at[pl.ds(r0, RPT)], cacc.at[pl.ds(r0, RPT)])
    o = jnp.ones((16,), jnp.float32)
    for k in range(CHUNK // 16):
        ones_v[pl.ds(k * 16, 16)] = o
    plsc.subcore_barrier()

    eb = [eb0, eb1]

    def cnt_scatter(q):
        pltpu.sync_copy(ones_v, cacc.at[eb[q].at[1]], add=True)

    _serial_edge_loop(x_hbm, ec_hbm, w * 2 * (EPAD // 32 // CHUNK) * 1,
                      nchunk, eb, rows_v, acc, [si0, si1], gsem,
                      extra=cnt_scatter)

    plsc.subcore_barrier()
    pltpu.sync_copy(acc.at[pl.ds(r0, RPT)],
                    agg_hbm.at[pl.ds(c * NP + r0, RPT)])
    pltpu.sync_copy(cacc.at[pl.ds(r0, RPT)],
                    cnt_hbm.at[pl.ds(c * NP + r0, RPT)])


_sc_agg0 = pl.kernel(
    _agg0_body,
    out_type=(jax.ShapeDtypeStruct((2 * NP, 128), jnp.float32),
              jax.ShapeDtypeStruct((2 * NP,), jnp.float32)),
    mesh=_mesh,
    scratch_types=[
        pltpu.VMEM((2, CHUNK), jnp.int32),
        pltpu.VMEM((2, CHUNK), jnp.int32),
        pltpu.VMEM((CHUNK, 128), jnp.float32),
        pltpu.VMEM((CHUNK,), jnp.float32),
        pltpu.VMEM_SHARED((NP, 128), jnp.float32),
        pltpu.VMEM_SHARED((NP,), jnp.float32),
        pltpu.SemaphoreType.DMA,
        pltpu.SemaphoreType.DMA,
        pltpu.SemaphoreType.DMA,
    ],
)


def _agg_body(h_hbm, ec_hbm, z2d_hbm,
              agg_hbm,
              eb0, eb1, rows_v, acc, si0, si1, gsem):
    c = lax.axis_index("c")
    s = lax.axis_index("s")
    ept = EPAD // 16
    nchunk = ept // CHUNK

    r0 = s * RPT
    pltpu.sync_copy(z2d_hbm.at[pl.ds(r0, RPT)], acc.at[pl.ds(r0, RPT)])
    plsc.subcore_barrier()

    erow0 = (c * 16 + s) * 2 * nchunk
    _serial_edge_loop(h_hbm, ec_hbm, erow0, nchunk,
                      [eb0, eb1], rows_v, acc, [si0, si1], gsem)

    plsc.subcore_barrier()
    pltpu.sync_copy(acc.at[pl.ds(r0, RPT)],
                    agg_hbm.at[pl.ds(c * NP + r0, RPT)])


_sc_agg = pl.kernel(
    _agg_body,
    out_type=jax.ShapeDtypeStruct((2 * NP, 128), jnp.float32),
    mesh=_mesh,
    scratch_types=[
        pltpu.VMEM((2, CHUNK), jnp.int32),
        pltpu.VMEM((2, CHUNK), jnp.int32),
        pltpu.VMEM((CHUNK, 128), jnp.float32),
        pltpu.VMEM_SHARED((NP, 128), jnp.float32),
        pltpu.SemaphoreType.DMA,
        pltpu.SemaphoreType.DMA,
        pltpu.SemaphoreType.DMA,
    ],
)


ROW_BLK = 2048


def _tc_layer_body(nparts_in, parts_out,
                   agg_ref, cnt_ref, h_ref, wl_ref, wr_ref, b_ref, o_ref):
    cnt = cnt_ref[0] + cnt_ref[1]
    inv = 1.0 / jnp.maximum(cnt, 1.0)
    acc = jnp.zeros((ROW_BLK, 128), jnp.float32)
    for c in range(2):
        acc = acc + jnp.dot(agg_ref[c] * inv[:, None], wl_ref[c],
                            preferred_element_type=jnp.float32)
    for q in range(nparts_in):
        acc = acc + jnp.dot(h_ref[q], wr_ref[q],
                            preferred_element_type=jnp.float32)
    acc = acc + b_ref[0][None, :]
    out = jnp.maximum(acc, 0.0)
    if parts_out:
        o_ref[...] = out[None]
    else:
        o_ref[...] = out


def _tc_layer(agg, cnt, h_parts, wlt, wrt, b, parts_out):
    p_in = h_parts.shape[0]
    grid = (5, 2)
    if parts_out:
        out_shape = jax.ShapeDtypeStruct((2, NP, 128), jnp.float32)
        out_spec = pl.BlockSpec((1, ROW_BLK, 128), lambda i, p: (p, i, 0))
    else:
        out_shape = jax.ShapeDtypeStruct((N, D), jnp.float32)
        out_spec = pl.BlockSpec((ROW_BLK, 128), lambda i, p: (i, p))
    return pl.pallas_call(
        functools.partial(_tc_layer_body, p_in, parts_out),
        grid=grid,
        in_specs=[
            pl.BlockSpec((2, ROW_BLK, 128), lambda i, p: (0, i, 0)),
            pl.BlockSpec((2, ROW_BLK), lambda i, p: (0, i)),
            pl.BlockSpec((p_in, ROW_BLK, 128), lambda i, p: (0, i, 0)),
            pl.BlockSpec((2, 128, 128), lambda i, p: (0, 0, p)),
            pl.BlockSpec((p_in, 128, 128), lambda i, p: (0, 0, p)),
            pl.BlockSpec((1, 128), lambda i, p: (0, p)),
        ],
        out_specs=out_spec,
        out_shape=out_shape,
    )(agg, cnt, h_parts, wlt, wrt, b)


def kernel(x, edge_index, Wl0, Wr0, b0, Wl1, Wr1, b1, Wl2, Wr2, b2):
    src = edge_index[0]
    dst = edge_index[1]
    npad = EPAD - E
    src_pad = jnp.concatenate([src, jnp.zeros((npad,), jnp.int32)])
    dst_pad = jnp.concatenate([dst, jnp.full((npad,), NP - 1, jnp.int32)])
    src2d = src_pad.reshape(EPAD // CHUNK, CHUNK)
    dst2d = dst_pad.reshape(EPAD // CHUNK, CHUNK)
    pad2 = jnp.zeros((2, CHUNK), jnp.int32)
    ec_a = jnp.concatenate(
        [jnp.stack([src2d, dst2d], axis=1).reshape(-1, CHUNK), pad2])
    ec_b = jnp.concatenate(
        [jnp.stack([src2d, dst2d], axis=1).reshape(-1, CHUNK),
         jnp.stack([src2d + NP, dst2d], axis=1).reshape(-1, CHUNK), pad2])
    z2d = jnp.zeros((NP, 128), jnp.float32)
    z1d = jnp.zeros((NP,), jnp.float32)

    agg0, cnt = _sc_agg0(x, ec_a, z2d, z1d)
    agg0 = agg0.reshape(2, NP, 128)
    cnt = cnt.reshape(2, NP)
    h1 = _tc_layer(agg0, cnt, x.reshape(1, N, 128),
                   jnp.stack([Wl0.T, Wl0.T]), Wr0.T.reshape(1, 128, D),
                   b0.reshape(1, D), parts_out=True)

    agg1 = _sc_agg(h1.reshape(2 * NP, 128), ec_b, z2d)
    h2 = _tc_layer(agg1.reshape(2, NP, 128), cnt, h1,
                   Wl1.T.reshape(2, 128, D), Wr1.T.reshape(2, 128, D),
                   b1.reshape(1, D), parts_out=True)

    agg2 = _sc_agg(h2.reshape(2 * NP, 128), ec_b, z2d)
    h3 = _tc_layer(agg2.reshape(2, NP, 128), cnt, h2,
                   Wl2.T.reshape(2, 128, D), Wr2.T.reshape(2, 128, D),
                   b2.reshape(1, D), parts_out=False)

    return h3.reshape(1, N, D)


# speedup vs baseline: 1.0683x; 1.0683x over previous
import functools

import jax
import jax.numpy as jnp
from jax import lax
from jax.experimental import pallas as pl
from jax.experimental.pallas import tpu as pltpu
from jax.experimental.pallas import tpu_sc as plsc

N = 10000
E = 320000
D_IN = 128
D = 256
NP = 10240
RPT = NP // 16
CHUNK = 128
EPAD = 327680

_mesh = plsc.VectorSubcoreMesh(core_axis_name="c", subcore_axis_name="s")


def _serial_edge_loop(table_hbm, src_hbm, dst_hbm, sbase, dbase, nchunk,
                      idxv, dstv, rows_v, acc, six, sid, gsem, extra=None):
    def isx(t, q):
        pltpu.async_copy(src_hbm.at[pl.ds(sbase + t * CHUNK, CHUNK)],
                         idxv[q], six[q])

    def iwx(q):
        pltpu.make_async_copy(src_hbm.at[pl.ds(0, CHUNK)], idxv[q],
                              six[q]).wait()

    def isd(t, q):
        pltpu.async_copy(dst_hbm.at[pl.ds(dbase + t * CHUNK, CHUNK)],
                         dstv[q], sid[q])

    def iwd(q):
        pltpu.make_async_copy(dst_hbm.at[pl.ds(0, CHUNK)], dstv[q],
                              sid[q]).wait()

    isx(0, 0)
    isd(0, 0)

    def body(t2, _):
        for u in range(2):
            t = 2 * t2 + u
            q = u
            iwx(q)
            iwd(q)
            isx(t + 1, 1 - q)
            isd(t + 1, 1 - q)
            pltpu.async_copy(table_hbm.at[idxv[q]], rows_v, gsem).wait()
            pltpu.sync_copy(rows_v, acc.at[dstv[q]], add=True)
            if extra is not None:
                extra(q)
        return 0

    lax.fori_loop(0, nchunk // 2, body, 0)
    iwx(0)
    iwd(0)


def _agg0_body(x_hbm, src_hbm, dst_hbm, z2d_hbm, z1d_hbm,
               agg_hbm, cnt_hbm,
               ix0, ix1, id0, id1, rows_v, ones_v, acc, cacc,
               six0, six1, sid0, sid1, gsem):
    c = lax.axis_index("c")
    s = lax.axis_index("s")
    w = c * 16 + s
    nchunk = EPAD // 32 // CHUNK

    r0 = s * RPT
    pltpu.sync_copy(z2d_hbm.at[pl.ds(r0, RPT)], acc.at[pl.ds(r0, RPT)])
    pltpu.sync_copy(z1d_hbm.at[pl.ds(r0, RPT)], cacc.at[pl.ds(r0, RPT)])
    o = jnp.ones((16,), jnp.float32)
    for k in range(CHUNK // 16):
        ones_v[pl.ds(k * 16, 16)] = o
    plsc.subcore_barrier()

    dstv = [id0, id1]

    def cnt_scatter(q):
        pltpu.sync_copy(ones_v, cacc.at[dstv[q]], add=True)

    base = w * (EPAD // 32)
    _serial_edge_loop(x_hbm, src_hbm, dst_hbm, base, base, nchunk,
                      [ix0, ix1], dstv, rows_v, acc,
                      [six0, six1], [sid0, sid1], gsem, extra=cnt_scatter)

    plsc.subcore_barrier()
    pltpu.sync_copy(acc.at[pl.ds(r0, RPT)],
                    agg_hbm.at[pl.ds(c * NP + r0, RPT)])
    pltpu.sync_copy(cacc.at[pl.ds(r0, RPT)],
                    cnt_hbm.at[pl.ds(c * NP + r0, RPT)])


_sc_agg0 = pl.kernel(
    _agg0_body,
    out_type=(jax.ShapeDtypeStruct((2 * NP, 128), jnp.float32),
              jax.ShapeDtypeStruct((2 * NP,), jnp.float32)),
    mesh=_mesh,
    scratch_types=[
        pltpu.VMEM((CHUNK,), jnp.int32),
        pltpu.VMEM((CHUNK,), jnp.int32),
        pltpu.VMEM((CHUNK,), jnp.int32),
        pltpu.VMEM((CHUNK,), jnp.int32),
        pltpu.VMEM((CHUNK, 128), jnp.float32),
        pltpu.VMEM((CHUNK,), jnp.float32),
        pltpu.VMEM_SHARED((NP, 128), jnp.float32),
        pltpu.VMEM_SHARED((NP,), jnp.float32),
    ] + [pltpu.SemaphoreType.DMA] * 5,
)


def _agg_body(h_hbm, srcb_hbm, dst_hbm, z2d_hbm,
              agg_hbm,
              ix0, ix1, id0, id1, rows_v, acc,
              six0, six1, sid0, sid1, gsem):
    c = lax.axis_index("c")
    s = lax.axis_index("s")
    ept = EPAD // 16
    nchunk = ept // CHUNK

    r0 = s * RPT
    pltpu.sync_copy(z2d_hbm.at[pl.ds(r0, RPT)], acc.at[pl.ds(r0, RPT)])
    plsc.subcore_barrier()

    _serial_edge_loop(h_hbm, srcb_hbm, dst_hbm,
                      c * EPAD + s * ept, s * ept, nchunk,
                      [ix0, ix1], [id0, id1], rows_v, acc,
                      [six0, six1], [sid0, sid1], gsem)

    plsc.subcore_barrier()
    pltpu.sync_copy(acc.at[pl.ds(r0, RPT)],
                    agg_hbm.at[pl.ds(c * NP + r0, RPT)])


_sc_agg = pl.kernel(
    _agg_body,
    out_type=jax.ShapeDtypeStruct((2 * NP, 128), jnp.float32),
    mesh=_mesh,
    scratch_types=[
        pltpu.VMEM((CHUNK,), jnp.int32),
        pltpu.VMEM((CHUNK,), jnp.int32),
        pltpu.VMEM((CHUNK,), jnp.int32),
        pltpu.VMEM((CHUNK,), jnp.int32),
        pltpu.VMEM((CHUNK, 128), jnp.float32),
        pltpu.VMEM_SHARED((NP, 128), jnp.float32),
    ] + [pltpu.SemaphoreType.DMA] * 5,
)


ROW_BLK = 2048


def _tc_layer_body(nparts_in, parts_out,
                   agg_ref, cnt_ref, h_ref, wl_ref, wr_ref, b_ref, o_ref):
    cnt = cnt_ref[0] + cnt_ref[1]
    inv = 1.0 / jnp.maximum(cnt, 1.0)
    acc = jnp.zeros((ROW_BLK, 128), jnp.float32)
    for c in range(2):
        acc = acc + jnp.dot(agg_ref[c] * inv[:, None], wl_ref[c],
                            preferred_element_type=jnp.float32)
    for q in range(nparts_in):
        acc = acc + jnp.dot(h_ref[q], wr_ref[q],
                            preferred_element_type=jnp.float32)
    acc = acc + b_ref[0][None, :]
    out = jnp.maximum(acc, 0.0)
    if parts_out:
        o_ref[...] = out[None]
    else:
        o_ref[...] = out


def _tc_layer(agg, cnt, h_parts, wlt, wrt, b, parts_out):
    p_in = h_parts.shape[0]
    grid = (5, 2)
    if parts_out:
        out_shape = jax.ShapeDtypeStruct((2, NP, 128), jnp.float32)
        out_spec = pl.BlockSpec((1, ROW_BLK, 128), lambda i, p: (p, i, 0))
    else:
        out_shape = jax.ShapeDtypeStruct((N, D), jnp.float32)
        out_spec = pl.BlockSpec((ROW_BLK, 128), lambda i, p: (i, p))
    return pl.pallas_call(
        functools.partial(_tc_layer_body, p_in, parts_out),
        grid=grid,
        in_specs=[
            pl.BlockSpec((2, ROW_BLK, 128), lambda i, p: (0, i, 0)),
            pl.BlockSpec((2, ROW_BLK), lambda i, p: (0, i)),
            pl.BlockSpec((p_in, ROW_BLK, 128), lambda i, p: (0, i, 0)),
            pl.BlockSpec((2, 128, 128), lambda i, p: (0, 0, p)),
            pl.BlockSpec((p_in, 128, 128), lambda i, p: (0, 0, p)),
            pl.BlockSpec((1, 128), lambda i, p: (0, p)),
        ],
        out_specs=out_spec,
        out_shape=out_shape,
    )(agg, cnt, h_parts, wlt, wrt, b)


def kernel(x, edge_index, Wl0, Wr0, b0, Wl1, Wr1, b1, Wl2, Wr2, b2):
    src = edge_index[0]
    dst = edge_index[1]
    npad = EPAD - E + CHUNK
    src_pad = jnp.concatenate([src, jnp.zeros((npad,), jnp.int32)])
    dst_pad = jnp.concatenate([dst, jnp.full((npad,), NP - 1, jnp.int32)])
    srcb = jnp.concatenate([src[:1] * 0 + src_pad[:EPAD],
                            src_pad[:EPAD] + NP, jnp.zeros((CHUNK,), jnp.int32)])
    z2d = jnp.zeros((NP, 128), jnp.float32)
    z1d = jnp.zeros((NP,), jnp.float32)

    agg0, cnt = _sc_agg0(x, src_pad, dst_pad, z2d, z1d)
    agg0 = agg0.reshape(2, NP, 128)
    cnt = cnt.reshape(2, NP)
    h1 = _tc_layer(agg0, cnt, x.reshape(1, N, 128),
                   jnp.stack([Wl0.T, Wl0.T]), Wr0.T.reshape(1, 128, D),
                   b0.reshape(1, D), parts_out=True)

    agg1 = _sc_agg(h1.reshape(2 * NP, 128), srcb, dst_pad, z2d)
    h2 = _tc_layer(agg1.reshape(2, NP, 128), cnt, h1,
                   Wl1.T.reshape(2, 128, D), Wr1.T.reshape(2, 128, D),
                   b1.reshape(1, D), parts_out=True)

    agg2 = _sc_agg(h2.reshape(2 * NP, 128), srcb, dst_pad, z2d)
    h3 = _tc_layer(agg2.reshape(2, NP, 128), cnt, h2,
                   Wl2.T.reshape(2, 128, D), Wr2.T.reshape(2, 128, D),
                   b2.reshape(1, D), parts_out=False)

    return h3.reshape(1, N, D)


# speedup vs baseline: 1.6682x; 1.5615x over previous
import functools

import jax
import jax.numpy as jnp
from jax import lax
from jax.experimental import pallas as pl
from jax.experimental.pallas import tpu as pltpu
from jax.experimental.pallas import tpu_sc as plsc

N = 10000
E = 320000
D_IN = 128
D = 256
NP = 10240
ROWS_PER_TILE = NP // 16
CHUNK = 128

_mesh = plsc.VectorSubcoreMesh(core_axis_name="c", subcore_axis_name="s")


def _zero_block(zb):
    z = jnp.zeros((16,), jnp.float32)
    for r in range(16):
        for k in range(8):
            zb[r, pl.ds(k * 16, 16)] = z


def _zero_shared(zb, acc, s):
    def body(k, _):
        pltpu.sync_copy(zb, acc.at[pl.ds(s * ROWS_PER_TILE + k * 16, 16)])
        return 0
    lax.fori_loop(0, ROWS_PER_TILE // 16, body, 0)


def _agg0_body(x_hbm, src_hbm, dst_hbm, agg_hbm, cnt_hbm,
               idx_v, dst_v, rows_v, ones_v, idx_t, dst_t, rows_t,
               zb, zc, acc, cacc, sem):
    c = lax.axis_index("c")
    s = lax.axis_index("s")

    _zero_block(zb)
    _zero_shared(zb, acc, s)
    z = jnp.zeros((16,), jnp.float32)
    o = jnp.ones((16,), jnp.float32)
    for k in range(ROWS_PER_TILE // 16):
        zc[pl.ds(k * 16, 16)] = z
    for k in range(CHUNK // 16):
        ones_v[pl.ds(k * 16, 16)] = o
    pltpu.sync_copy(zc, cacc.at[pl.ds(s * ROWS_PER_TILE, ROWS_PER_TILE)])
    plsc.subcore_barrier()

    base_e = (c * 16 + s) * (E // 32)

    def body(t, _):
        off = base_e + t * CHUNK
        pltpu.sync_copy(src_hbm.at[pl.ds(off, CHUNK)], idx_v)
        pltpu.sync_copy(dst_hbm.at[pl.ds(off, CHUNK)], dst_v)
        pltpu.async_copy(x_hbm.at[idx_v], rows_v, sem).wait()
        pltpu.sync_copy(rows_v, acc.at[dst_v], add=True)
        pltpu.sync_copy(ones_v, cacc.at[dst_v], add=True)
        return 0

    lax.fori_loop(0, (E // 32) // CHUNK, body, 0)

    off = base_e + ((E // 32) // CHUNK) * CHUNK
    pltpu.sync_copy(src_hbm.at[pl.ds(off, 16)], idx_t)
    pltpu.sync_copy(dst_hbm.at[pl.ds(off, 16)], dst_t)
    pltpu.async_copy(x_hbm.at[idx_t], rows_t, sem).wait()
    pltpu.sync_copy(rows_t, acc.at[dst_t], add=True)
    pltpu.sync_copy(ones_v.at[pl.ds(0, 16)], cacc.at[dst_t], add=True)

    plsc.subcore_barrier()
    r0 = s * ROWS_PER_TILE
    pltpu.sync_copy(acc.at[pl.ds(r0, ROWS_PER_TILE)],
                    agg_hbm.at[pl.ds(c * NP + r0, ROWS_PER_TILE)])
    pltpu.sync_copy(cacc.at[pl.ds(r0, ROWS_PER_TILE)],
                    cnt_hbm.at[pl.ds(c * NP + r0, ROWS_PER_TILE)])


_sc_agg0 = pl.kernel(
    _agg0_body,
    out_type=(jax.ShapeDtypeStruct((2 * NP, 128), jnp.float32),
              jax.ShapeDtypeStruct((2 * NP,), jnp.float32)),
    mesh=_mesh,
    scratch_types=[
        pltpu.VMEM((CHUNK,), jnp.int32),
        pltpu.VMEM((CHUNK,), jnp.int32),
        pltpu.VMEM((CHUNK, 128), jnp.float32),
        pltpu.VMEM((CHUNK,), jnp.float32),
        pltpu.VMEM((16,), jnp.int32),
        pltpu.VMEM((16,), jnp.int32),
        pltpu.VMEM((16, 128), jnp.float32),
        pltpu.VMEM((16, 128), jnp.float32),
        pltpu.VMEM((ROWS_PER_TILE,), jnp.float32),
        pltpu.VMEM_SHARED((NP, 128), jnp.float32),
        pltpu.VMEM_SHARED((NP,), jnp.float32),
        pltpu.SemaphoreType.DMA,
    ],
)


def _agg_body(h_hbm, src_hbm, dst_hbm, agg_hbm,
              idx_v, dst_v, rows_v, idx_t, dst_t, rows_t,
              zb, acc, sem):
    c = lax.axis_index("c")
    s = lax.axis_index("s")

    _zero_block(zb)
    _zero_shared(zb, acc, s)
    plsc.subcore_barrier()

    base_e = s * (E // 16)

    def body(t, _):
        off = c * E + base_e + t * CHUNK
        doff = base_e + t * CHUNK
        pltpu.sync_copy(src_hbm.at[pl.ds(off, CHUNK)], idx_v)
        pltpu.sync_copy(dst_hbm.at[pl.ds(doff, CHUNK)], dst_v)
        pltpu.async_copy(h_hbm.at[idx_v], rows_v, sem).wait()
        pltpu.sync_copy(rows_v, acc.at[dst_v], add=True)
        return 0

    lax.fori_loop(0, (E // 16) // CHUNK, body, 0)

    toff = base_e + ((E // 16) // CHUNK) * CHUNK
    pltpu.sync_copy(src_hbm.at[pl.ds(c * E + toff, 32)], idx_t)
    pltpu.sync_copy(dst_hbm.at[pl.ds(toff, 32)], dst_t)
    pltpu.async_copy(h_hbm.at[idx_t], rows_t, sem).wait()
    pltpu.sync_copy(rows_t, acc.at[dst_t], add=True)

    plsc.subcore_barrier()
    r0 = s * ROWS_PER_TILE
    pltpu.sync_copy(acc.at[pl.ds(r0, ROWS_PER_TILE)],
                    agg_hbm.at[pl.ds(c * NP + r0, ROWS_PER_TILE)])


_sc_agg = pl.kernel(
    _agg_body,
    out_type=jax.ShapeDtypeStruct((2 * NP, 128), jnp.float32),
    mesh=_mesh,
    scratch_types=[
        pltpu.VMEM((CHUNK,), jnp.int32),
        pltpu.VMEM((CHUNK,), jnp.int32),
        pltpu.VMEM((CHUNK, 128), jnp.float32),
        pltpu.VMEM((32,), jnp.int32),
        pltpu.VMEM((32,), jnp.int32),
        pltpu.VMEM((32, 128), jnp.float32),
        pltpu.VMEM((16, 128), jnp.float32),
        pltpu.VMEM_SHARED((NP, 128), jnp.float32),
        pltpu.SemaphoreType.DMA,
    ],
)


ROW_BLK = 2048


def _tc_layer_body(nparts_in, parts_out,
                   agg_ref, cnt_ref, h_ref, wl_ref, wr_ref, b_ref, o_ref):
    cnt = cnt_ref[0] + cnt_ref[1]
    inv = 1.0 / jnp.maximum(cnt, 1.0)
    acc = jnp.zeros((ROW_BLK, 128), jnp.float32)
    for c in range(2):
        acc = acc + jnp.dot(agg_ref[c] * inv[:, None], wl_ref[c],
                            preferred_element_type=jnp.float32)
    for q in range(nparts_in):
        acc = acc + jnp.dot(h_ref[q], wr_ref[q],
                            preferred_element_type=jnp.float32)
    acc = acc + b_ref[0][None, :]
    out = jnp.maximum(acc, 0.0)
    if parts_out:
        o_ref[...] = out[None]
    else:
        o_ref[...] = out


def _tc_layer(agg, cnt, h_parts, wlt, wrt, b, parts_out):
    p_in = h_parts.shape[0]
    grid = (5, 2)
    if parts_out:
        out_shape = jax.ShapeDtypeStruct((2, NP, 128), jnp.float32)
        out_spec = pl.BlockSpec((1, ROW_BLK, 128), lambda i, p: (p, i, 0))
    else:
        out_shape = jax.ShapeDtypeStruct((N, D), jnp.float32)
        out_spec = pl.BlockSpec((ROW_BLK, 128), lambda i, p: (i, p))
    return pl.pallas_call(
        functools.partial(_tc_layer_body, p_in, parts_out),
        grid=grid,
        in_specs=[
            pl.BlockSpec((2, ROW_BLK, 128), lambda i, p: (0, i, 0)),
            pl.BlockSpec((2, ROW_BLK), lambda i, p: (0, i)),
            pl.BlockSpec((p_in, ROW_BLK, 128), lambda i, p: (0, i, 0)),
            pl.BlockSpec((2, 128, 128), lambda i, p: (0, 0, p)),
            pl.BlockSpec((p_in, 128, 128), lambda i, p: (0, 0, p)),
            pl.BlockSpec((1, 128), lambda i, p: (0, p)),
        ],
        out_specs=out_spec,
        out_shape=out_shape,
    )(agg, cnt, h_parts, wlt, wrt, b)


def kernel(x, edge_index, Wl0, Wr0, b0, Wl1, Wr1, b1, Wl2, Wr2, b2):
    src = edge_index[0]
    dst = edge_index[1]
    src_both = jnp.concatenate([src, src + NP])

    agg0, cnt = _sc_agg0(x, src, dst)
    agg0 = agg0.reshape(2, NP, 128)
    cnt = cnt.reshape(2, NP)
    h1 = _tc_layer(agg0, cnt, x.reshape(1, N, 128),
                   jnp.stack([Wl0.T, Wl0.T]), Wr0.T.reshape(1, 128, D),
                   b0.reshape(1, D), parts_out=True)

    agg1 = _sc_agg(h1.reshape(2 * NP, 128), src_both, dst).reshape(2, NP, 128)
    h2 = _tc_layer(agg1, cnt, h1,
                   Wl1.T.reshape(2, 128, D), Wr1.T.reshape(2, 128, D),
                   b1.reshape(1, D), parts_out=True)

    agg2 = _sc_agg(h2.reshape(2 * NP, 128), src_both, dst).reshape(2, NP, 128)
    h3 = _tc_layer(agg2, cnt, h2,
                   Wl2.T.reshape(2, 128, D), Wr2.T.reshape(2, 128, D),
                   b2.reshape(1, D), parts_out=False)

    return h3.reshape(1, N, D)


# speedup vs baseline: 1.9378x; 1.1616x over previous
import functools

import jax
import jax.numpy as jnp
from jax import lax
from jax.experimental import pallas as pl
from jax.experimental.pallas import tpu as pltpu
from jax.experimental.pallas import tpu_sc as plsc

N = 10000
E = 320000
D_IN = 128
D = 256
NP = 10240
ROWS_PER_TILE = NP // 16
CHUNK = 128

_mesh = plsc.VectorSubcoreMesh(core_axis_name="c", subcore_axis_name="s")


def _zero_block(zb):
    z = jnp.zeros((16,), jnp.float32)
    for r in range(16):
        for k in range(8):
            zb[r, pl.ds(k * 16, 16)] = z


def _zero_shared(zb, acc, s):
    def body(k, _):
        pltpu.sync_copy(zb, acc.at[pl.ds(s * ROWS_PER_TILE + k * 16, 16)])
        return 0
    lax.fori_loop(0, ROWS_PER_TILE // 16, body, 0)


def _agg0_body(x_hbm, src_hbm, dst_hbm, agg_hbm, cnt_hbm,
               idx_v, dst_v, rows_v, ones_v, idx_t, dst_t, rows_t,
               zb, zc, acc, cacc, sem):
    c = lax.axis_index("c")
    s = lax.axis_index("s")

    _zero_block(zb)
    _zero_shared(zb, acc, s)
    z = jnp.zeros((16,), jnp.float32)
    o = jnp.ones((16,), jnp.float32)
    for k in range(ROWS_PER_TILE // 16):
        zc[pl.ds(k * 16, 16)] = z
    for k in range(CHUNK // 16):
        ones_v[pl.ds(k * 16, 16)] = o
    pltpu.sync_copy(zc, cacc.at[pl.ds(s * ROWS_PER_TILE, ROWS_PER_TILE)])
    plsc.subcore_barrier()

    base_e = (c * 16 + s) * (E // 32)

    def body(t, _):
        off = base_e + t * CHUNK
        pltpu.sync_copy(src_hbm.at[pl.ds(off, CHUNK)], idx_v)
        gather = pltpu.async_copy(x_hbm.at[idx_v], rows_v, sem)
        pltpu.sync_copy(dst_hbm.at[pl.ds(off, CHUNK)], dst_v)
        gather.wait()
        pltpu.sync_copy(rows_v, acc.at[dst_v], add=True)
        pltpu.sync_copy(ones_v, cacc.at[dst_v], add=True)
        return 0

    lax.fori_loop(0, (E // 32) // CHUNK, body, 0)

    off = base_e + ((E // 32) // CHUNK) * CHUNK
    pltpu.sync_copy(src_hbm.at[pl.ds(off, 16)], idx_t)
    pltpu.sync_copy(dst_hbm.at[pl.ds(off, 16)], dst_t)
    pltpu.async_copy(x_hbm.at[idx_t], rows_t, sem).wait()
    pltpu.sync_copy(rows_t, acc.at[dst_t], add=True)
    pltpu.sync_copy(ones_v.at[pl.ds(0, 16)], cacc.at[dst_t], add=True)

    plsc.subcore_barrier()
    r0 = s * ROWS_PER_TILE
    pltpu.sync_copy(acc.at[pl.ds(r0, ROWS_PER_TILE)],
                    agg_hbm.at[pl.ds(c * NP + r0, ROWS_PER_TILE)])
    pltpu.sync_copy(cacc.at[pl.ds(r0, ROWS_PER_TILE)],
                    cnt_hbm.at[pl.ds(c * NP + r0, ROWS_PER_TILE)])


_sc_agg0 = pl.kernel(
    _agg0_body,
    out_type=(jax.ShapeDtypeStruct((2 * NP, 128), jnp.float32),
              jax.ShapeDtypeStruct((2 * NP,), jnp.float32)),
    mesh=_mesh,
    scratch_types=[
        pltpu.VMEM((CHUNK,), jnp.int32),
        pltpu.VMEM((CHUNK,), jnp.int32),
        pltpu.VMEM((CHUNK, 128), jnp.float32),
        pltpu.VMEM((CHUNK,), jnp.float32),
        pltpu.VMEM((16,), jnp.int32),
        pltpu.VMEM((16,), jnp.int32),
        pltpu.VMEM((16, 128), jnp.float32),
        pltpu.VMEM((16, 128), jnp.float32),
        pltpu.VMEM((ROWS_PER_TILE,), jnp.float32),
        pltpu.VMEM_SHARED((NP, 128), jnp.float32),
        pltpu.VMEM_SHARED((NP,), jnp.float32),
        pltpu.SemaphoreType.DMA,
    ],
)


def _agg_body(h_hbm, src_hbm, dst_hbm, agg_hbm,
              idx_v, dst_v, rows_v, idx_t, dst_t, rows_t,
              zb, acc, sem):
    c = lax.axis_index("c")
    s = lax.axis_index("s")

    _zero_block(zb)
    _zero_shared(zb, acc, s)
    plsc.subcore_barrier()

    base_e = s * (E // 16)

    def body(t, _):
        off = c * E + base_e + t * CHUNK
        doff = base_e + t * CHUNK
        pltpu.sync_copy(src_hbm.at[pl.ds(off, CHUNK)], idx_v)
        gather = pltpu.async_copy(h_hbm.at[idx_v], rows_v, sem)
        pltpu.sync_copy(dst_hbm.at[pl.ds(doff, CHUNK)], dst_v)
        gather.wait()
        pltpu.sync_copy(rows_v, acc.at[dst_v], add=True)
        return 0

    lax.fori_loop(0, (E // 16) // CHUNK, body, 0)

    toff = base_e + ((E // 16) // CHUNK) * CHUNK
    pltpu.sync_copy(src_hbm.at[pl.ds(c * E + toff, 32)], idx_t)
    pltpu.sync_copy(dst_hbm.at[pl.ds(toff, 32)], dst_t)
    pltpu.async_copy(h_hbm.at[idx_t], rows_t, sem).wait()
    pltpu.sync_copy(rows_t, acc.at[dst_t], add=True)

    plsc.subcore_barrier()
    r0 = s * ROWS_PER_TILE
    pltpu.sync_copy(acc.at[pl.ds(r0, ROWS_PER_TILE)],
                    agg_hbm.at[pl.ds(c * NP + r0, ROWS_PER_TILE)])


_sc_agg = pl.kernel(
    _agg_body,
    out_type=jax.ShapeDtypeStruct((2 * NP, 128), jnp.float32),
    mesh=_mesh,
    scratch_types=[
        pltpu.VMEM((CHUNK,), jnp.int32),
        pltpu.VMEM((CHUNK,), jnp.int32),
        pltpu.VMEM((CHUNK, 128), jnp.float32),
        pltpu.VMEM((32,), jnp.int32),
        pltpu.VMEM((32,), jnp.int32),
        pltpu.VMEM((32, 128), jnp.float32),
        pltpu.VMEM((16, 128), jnp.float32),
        pltpu.VMEM_SHARED((NP, 128), jnp.float32),
        pltpu.SemaphoreType.DMA,
    ],
)


ROW_BLK = 2048


def _tc_layer_body(nparts_in, parts_out,
                   agg_ref, cnt_ref, h_ref, wl_ref, wr_ref, b_ref, o_ref):
    cnt = cnt_ref[0] + cnt_ref[1]
    inv = 1.0 / jnp.maximum(cnt, 1.0)
    acc = jnp.zeros((ROW_BLK, 128), jnp.float32)
    for c in range(2):
        acc = acc + jnp.dot(agg_ref[c] * inv[:, None], wl_ref[c],
                            preferred_element_type=jnp.float32)
    for q in range(nparts_in):
        acc = acc + jnp.dot(h_ref[q], wr_ref[q],
                            preferred_element_type=jnp.float32)
    acc = acc + b_ref[0][None, :]
    out = jnp.maximum(acc, 0.0)
    if parts_out:
        o_ref[...] = out[None]
    else:
        o_ref[...] = out


def _tc_layer(agg, cnt, h_parts, wlt, wrt, b, parts_out):
    p_in = h_parts.shape[0]
    grid = (5, 2)
    if parts_out:
        out_shape = jax.ShapeDtypeStruct((2, NP, 128), jnp.float32)
        out_spec = pl.BlockSpec((1, ROW_BLK, 128), lambda i, p: (p, i, 0))
    else:
        out_shape = jax.ShapeDtypeStruct((N, D), jnp.float32)
        out_spec = pl.BlockSpec((ROW_BLK, 128), lambda i, p: (i, p))
    return pl.pallas_call(
        functools.partial(_tc_layer_body, p_in, parts_out),
        grid=grid,
        in_specs=[
            pl.BlockSpec((2, ROW_BLK, 128), lambda i, p: (0, i, 0)),
            pl.BlockSpec((2, ROW_BLK), lambda i, p: (0, i)),
            pl.BlockSpec((p_in, ROW_BLK, 128), lambda i, p: (0, i, 0)),
            pl.BlockSpec((2, 128, 128), lambda i, p: (0, 0, p)),
            pl.BlockSpec((p_in, 128, 128), lambda i, p: (0, 0, p)),
            pl.BlockSpec((1, 128), lambda i, p: (0, p)),
        ],
        out_specs=out_spec,
        out_shape=out_shape,
    )(agg, cnt, h_parts, wlt, wrt, b)


def kernel(x, edge_index, Wl0, Wr0, b0, Wl1, Wr1, b1, Wl2, Wr2, b2):
    src = edge_index[0]
    dst = edge_index[1]
    src_both = jnp.concatenate([src, src + NP])

    agg0, cnt = _sc_agg0(x, src, dst)
    agg0 = agg0.reshape(2, NP, 128)
    cnt = cnt.reshape(2, NP)
    h1 = _tc_layer(agg0, cnt, x.reshape(1, N, 128),
                   jnp.stack([Wl0.T, Wl0.T]), Wr0.T.reshape(1, 128, D),
                   b0.reshape(1, D), parts_out=True)

    agg1 = _sc_agg(h1.reshape(2 * NP, 128), src_both, dst).reshape(2, NP, 128)
    h2 = _tc_layer(agg1, cnt, h1,
                   Wl1.T.reshape(2, 128, D), Wr1.T.reshape(2, 128, D),
                   b1.reshape(1, D), parts_out=True)

    agg2 = _sc_agg(h2.reshape(2 * NP, 128), src_both, dst).reshape(2, NP, 128)
    h3 = _tc_layer(agg2, cnt, h2,
                   Wl2.T.reshape(2, 128, D), Wr2.T.reshape(2, 128, D),
                   b2.reshape(1, D), parts_out=False)

    return h3.reshape(1, N, D)


# speedup vs baseline: 2.4845x; 1.2822x over previous
import functools

import jax
import jax.numpy as jnp
from jax import lax
from jax.experimental import pallas as pl
from jax.experimental.pallas import tpu as pltpu
from jax.experimental.pallas import tpu_sc as plsc

N = 10000
E = 320000
D_IN = 128
D = 256
NP = 10240
ROWS_PER_TILE = NP // 16
CHUNK = 128

_mesh = plsc.VectorSubcoreMesh(core_axis_name="c", subcore_axis_name="s")


def _zero_block(zb):
    z = jnp.zeros((16,), jnp.float32)
    for r in range(16):
        for k in range(8):
            zb[r, pl.ds(k * 16, 16)] = z


def _zero_shared(zb, acc, s):
    def body(k, _):
        pltpu.sync_copy(zb, acc.at[pl.ds(s * ROWS_PER_TILE + k * 16, 16)])
        return 0
    lax.fori_loop(0, ROWS_PER_TILE // 16, body, 0)


def _agg0_body(x_hbm, src_hbm, dst_hbm, agg_hbm, cnt_hbm,
               idx_v, dst_v, dst_v2, rows_v, rows_v2, ones_v,
               idx_t, dst_t, rows_t,
               zb, zc, acc, cacc, sem, ss0, ss1, cs0, cs1):
    c = lax.axis_index("c")
    s = lax.axis_index("s")

    _zero_block(zb)
    _zero_shared(zb, acc, s)
    z = jnp.zeros((16,), jnp.float32)
    o = jnp.ones((16,), jnp.float32)
    for k in range(ROWS_PER_TILE // 16):
        zc[pl.ds(k * 16, 16)] = z
    for k in range(CHUNK // 16):
        ones_v[pl.ds(k * 16, 16)] = o
    pltpu.sync_copy(zc, cacc.at[pl.ds(s * ROWS_PER_TILE, ROWS_PER_TILE)])
    plsc.subcore_barrier()

    base_e = (c * 16 + s) * (E // 32)
    rows = [rows_v, rows_v2]
    dstv = [dst_v, dst_v2]
    ssem = [ss0, ss1]
    csem = [cs0, cs1]

    def sw(u):
        pltpu.make_async_copy(rows[u], acc.at[dstv[u]], ssem[u]).wait()
        pltpu.make_async_copy(ones_v, cacc.at[dstv[u]], csem[u]).wait()

    def do_chunk(off, u, wait_prev):
        pltpu.sync_copy(src_hbm.at[pl.ds(off, CHUNK)], idx_v)
        if wait_prev:
            sw(u)
        gather = pltpu.async_copy(x_hbm.at[idx_v], rows[u], sem)
        pltpu.sync_copy(dst_hbm.at[pl.ds(off, CHUNK)], dstv[u])
        gather.wait()
        pltpu.async_copy(rows[u], acc.at[dstv[u]], ssem[u], add=True)
        pltpu.async_copy(ones_v, cacc.at[dstv[u]], csem[u], add=True)

    for u in range(2):
        do_chunk(base_e + u * CHUNK, u, False)

    def body(t2, _):
        for u in range(2):
            do_chunk(base_e + (2 + 2 * t2 + u) * CHUNK, u, True)
        return 0

    lax.fori_loop(0, ((E // 32) // CHUNK - 2) // 2, body, 0)
    sw(0)
    sw(1)

    off = base_e + ((E // 32) // CHUNK) * CHUNK
    pltpu.sync_copy(src_hbm.at[pl.ds(off, 16)], idx_t)
    pltpu.sync_copy(dst_hbm.at[pl.ds(off, 16)], dst_t)
    pltpu.async_copy(x_hbm.at[idx_t], rows_t, sem).wait()
    pltpu.sync_copy(rows_t, acc.at[dst_t], add=True)
    pltpu.sync_copy(ones_v.at[pl.ds(0, 16)], cacc.at[dst_t], add=True)

    plsc.subcore_barrier()
    r0 = s * ROWS_PER_TILE
    pltpu.sync_copy(acc.at[pl.ds(r0, ROWS_PER_TILE)],
                    agg_hbm.at[pl.ds(c * NP + r0, ROWS_PER_TILE)])
    pltpu.sync_copy(cacc.at[pl.ds(r0, ROWS_PER_TILE)],
                    cnt_hbm.at[pl.ds(c * NP + r0, ROWS_PER_TILE)])


_sc_agg0 = pl.kernel(
    _agg0_body,
    out_type=(jax.ShapeDtypeStruct((2 * NP, 128), jnp.float32),
              jax.ShapeDtypeStruct((2 * NP,), jnp.float32)),
    mesh=_mesh,
    scratch_types=[
        pltpu.VMEM((CHUNK,), jnp.int32),
        pltpu.VMEM((CHUNK,), jnp.int32),
        pltpu.VMEM((CHUNK,), jnp.int32),
        pltpu.VMEM((CHUNK, 128), jnp.float32),
        pltpu.VMEM((CHUNK, 128), jnp.float32),
        pltpu.VMEM((CHUNK,), jnp.float32),
        pltpu.VMEM((16,), jnp.int32),
        pltpu.VMEM((16,), jnp.int32),
        pltpu.VMEM((16, 128), jnp.float32),
        pltpu.VMEM((16, 128), jnp.float32),
        pltpu.VMEM((ROWS_PER_TILE,), jnp.float32),
        pltpu.VMEM_SHARED((NP, 128), jnp.float32),
        pltpu.VMEM_SHARED((NP,), jnp.float32),
    ] + [pltpu.SemaphoreType.DMA] * 5,
)


def _agg_body(h_hbm, src_hbm, dst_hbm, agg_hbm,
              idx_v, dst_v, dst_v2, rows_v, rows_v2, idx_t, dst_t, rows_t,
              zb, acc, sem, ss0, ss1):
    c = lax.axis_index("c")
    s = lax.axis_index("s")

    _zero_block(zb)
    _zero_shared(zb, acc, s)
    plsc.subcore_barrier()

    base_e = s * (E // 16)
    rows = [rows_v, rows_v2]
    dstv = [dst_v, dst_v2]
    ssem = [ss0, ss1]

    def sw(u):
        pltpu.make_async_copy(rows[u], acc.at[dstv[u]], ssem[u]).wait()

    def do_chunk(t, u, wait_prev):
        pltpu.sync_copy(src_hbm.at[pl.ds(c * E + base_e + t * CHUNK, CHUNK)],
                        idx_v)
        if wait_prev:
            sw(u)
        gather = pltpu.async_copy(h_hbm.at[idx_v], rows[u], sem)
        pltpu.sync_copy(dst_hbm.at[pl.ds(base_e + t * CHUNK, CHUNK)],
                        dstv[u])
        gather.wait()
        pltpu.async_copy(rows[u], acc.at[dstv[u]], ssem[u], add=True)

    for u in range(2):
        do_chunk(u, u, False)

    def body(t2, _):
        for u in range(2):
            do_chunk(2 + 2 * t2 + u, u, True)
        return 0

    lax.fori_loop(0, ((E // 16) // CHUNK - 2) // 2, body, 0)
    sw(0)
    sw(1)

    toff = base_e + ((E // 16) // CHUNK) * CHUNK
    pltpu.sync_copy(src_hbm.at[pl.ds(c * E + toff, 32)], idx_t)
    pltpu.sync_copy(dst_hbm.at[pl.ds(toff, 32)], dst_t)
    pltpu.async_copy(h_hbm.at[idx_t], rows_t, sem).wait()
    pltpu.sync_copy(rows_t, acc.at[dst_t], add=True)

    plsc.subcore_barrier()
    r0 = s * ROWS_PER_TILE
    pltpu.sync_copy(acc.at[pl.ds(r0, ROWS_PER_TILE)],
                    agg_hbm.at[pl.ds(c * NP + r0, ROWS_PER_TILE)])


_sc_agg = pl.kernel(
    _agg_body,
    out_type=jax.ShapeDtypeStruct((2 * NP, 128), jnp.float32),
    mesh=_mesh,
    scratch_types=[
        pltpu.VMEM((CHUNK,), jnp.int32),
        pltpu.VMEM((CHUNK,), jnp.int32),
        pltpu.VMEM((CHUNK,), jnp.int32),
        pltpu.VMEM((CHUNK, 128), jnp.float32),
        pltpu.VMEM((CHUNK, 128), jnp.float32),
        pltpu.VMEM((32,), jnp.int32),
        pltpu.VMEM((32,), jnp.int32),
        pltpu.VMEM((32, 128), jnp.float32),
        pltpu.VMEM((16, 128), jnp.float32),
        pltpu.VMEM_SHARED((NP, 128), jnp.float32),
    ] + [pltpu.SemaphoreType.DMA] * 3,
)


ROW_BLK = 2048


def _tc_layer_body(nparts_in, parts_out,
                   agg_ref, cnt_ref, h_ref, wl_ref, wr_ref, b_ref, o_ref):
    cnt = cnt_ref[0] + cnt_ref[1]
    inv = 1.0 / jnp.maximum(cnt, 1.0)
    acc = jnp.zeros((ROW_BLK, 128), jnp.float32)
    for c in range(2):
        acc = acc + jnp.dot(agg_ref[c] * inv[:, None], wl_ref[c],
                            preferred_element_type=jnp.float32)
    for q in range(nparts_in):
        acc = acc + jnp.dot(h_ref[q], wr_ref[q],
                            preferred_element_type=jnp.float32)
    acc = acc + b_ref[0][None, :]
    out = jnp.maximum(acc, 0.0)
    if parts_out:
        o_ref[...] = out[None]
    else:
        o_ref[...] = out


def _tc_layer(agg, cnt, h_parts, wlt, wrt, b, parts_out):
    p_in = h_parts.shape[0]
    grid = (5, 2)
    if parts_out:
        out_shape = jax.ShapeDtypeStruct((2, NP, 128), jnp.float32)
        out_spec = pl.BlockSpec((1, ROW_BLK, 128), lambda i, p: (p, i, 0))
    else:
        out_shape = jax.ShapeDtypeStruct((N, D), jnp.float32)
        out_spec = pl.BlockSpec((ROW_BLK, 128), lambda i, p: (i, p))
    return pl.pallas_call(
        functools.partial(_tc_layer_body, p_in, parts_out),
        grid=grid,
        in_specs=[
            pl.BlockSpec((2, ROW_BLK, 128), lambda i, p: (0, i, 0)),
            pl.BlockSpec((2, ROW_BLK), lambda i, p: (0, i)),
            pl.BlockSpec((p_in, ROW_BLK, 128), lambda i, p: (0, i, 0)),
            pl.BlockSpec((2, 128, 128), lambda i, p: (0, 0, p)),
            pl.BlockSpec((p_in, 128, 128), lambda i, p: (0, 0, p)),
            pl.BlockSpec((1, 128), lambda i, p: (0, p)),
        ],
        out_specs=out_spec,
        out_shape=out_shape,
    )(agg, cnt, h_parts, wlt, wrt, b)


def kernel(x, edge_index, Wl0, Wr0, b0, Wl1, Wr1, b1, Wl2, Wr2, b2):
    src = edge_index[0]
    dst = edge_index[1]
    src_both = jnp.concatenate([src, src + NP])

    agg0, cnt = _sc_agg0(x, src, dst)
    agg0 = agg0.reshape(2, NP, 128)
    cnt = cnt.reshape(2, NP)
    h1 = _tc_layer(agg0, cnt, x.reshape(1, N, 128),
                   jnp.stack([Wl0.T, Wl0.T]), Wr0.T.reshape(1, 128, D),
                   b0.reshape(1, D), parts_out=True)

    agg1 = _sc_agg(h1.reshape(2 * NP, 128), src_both, dst).reshape(2, NP, 128)
    h2 = _tc_layer(agg1, cnt, h1,
                   Wl1.T.reshape(2, 128, D), Wr1.T.reshape(2, 128, D),
                   b1.reshape(1, D), parts_out=True)

    agg2 = _sc_agg(h2.reshape(2 * NP, 128), src_both, dst).reshape(2, NP, 128)
    h3 = _tc_layer(agg2, cnt, h2,
                   Wl2.T.reshape(2, 128, D), Wr2.T.reshape(2, 128, D),
                   b2.reshape(1, D), parts_out=False)

    return h3.reshape(1, N, D)


# speedup vs baseline: 3.0586x; 1.2311x over previous
import functools

import jax
import jax.numpy as jnp
from jax import lax
from jax.experimental import pallas as pl
from jax.experimental.pallas import tpu as pltpu
from jax.experimental.pallas import tpu_sc as plsc

N = 10000
E = 320000
D_IN = 128
D = 256
NP = 10240
ROWS_PER_TILE = NP // 16
CHUNK = 128

_mesh = plsc.VectorSubcoreMesh(core_axis_name="c", subcore_axis_name="s")


def _zero_block(zb):
    z = jnp.zeros((16,), jnp.float32)
    for r in range(16):
        for k in range(8):
            zb[r, pl.ds(k * 16, 16)] = z


def _zero_shared(zb, acc, s):
    def body(k, _):
        pltpu.sync_copy(zb, acc.at[pl.ds(s * ROWS_PER_TILE + k * 16, 16)])
        return 0
    lax.fori_loop(0, ROWS_PER_TILE // 16, body, 0)


def _agg0_body(x_hbm, src_hbm, dst_hbm, agg_hbm, cnt_hbm,
               idx_v, idx_v2, dst_v, dst_v2, rows_v, rows_v2, ones_v,
               idx_t, dst_t, rows_t,
               zb, zc, acc, cacc, sem, ss0, ss1, cs0, cs1, ix0, ix1):
    c = lax.axis_index("c")
    s = lax.axis_index("s")

    _zero_block(zb)
    _zero_shared(zb, acc, s)
    z = jnp.zeros((16,), jnp.float32)
    o = jnp.ones((16,), jnp.float32)
    for k in range(ROWS_PER_TILE // 16):
        zc[pl.ds(k * 16, 16)] = z
    for k in range(CHUNK // 16):
        ones_v[pl.ds(k * 16, 16)] = o
    pltpu.sync_copy(zc, cacc.at[pl.ds(s * ROWS_PER_TILE, ROWS_PER_TILE)])
    plsc.subcore_barrier()

    base_e = (c * 16 + s) * (E // 32)
    rows = [rows_v, rows_v2]
    dstv = [dst_v, dst_v2]
    idxv = [idx_v, idx_v2]
    ssem = [ss0, ss1]
    csem = [cs0, cs1]
    isem = [ix0, ix1]
    nch = (E // 32) // CHUNK

    def sw(u):
        pltpu.make_async_copy(rows[u], acc.at[dstv[u]], ssem[u]).wait()
        pltpu.make_async_copy(ones_v, cacc.at[dstv[u]], csem[u]).wait()

    def ipf(off, u):
        pltpu.async_copy(src_hbm.at[pl.ds(off, CHUNK)], idxv[u], isem[u])

    def ipw(u):
        pltpu.make_async_copy(src_hbm.at[pl.ds(0, CHUNK)], idxv[u],
                              isem[u]).wait()

    def do_chunk(off, u, wait_prev, pf_off):
        ipw(u)
        if wait_prev:
            sw(u)
        gather = pltpu.async_copy(x_hbm.at[idxv[u]], rows[u], sem)
        pltpu.sync_copy(dst_hbm.at[pl.ds(off, CHUNK)], dstv[u])
        gather.wait()
        if pf_off is not None:
            ipf(pf_off, u)
        pltpu.async_copy(rows[u], acc.at[dstv[u]], ssem[u], add=True)
        pltpu.async_copy(ones_v, cacc.at[dstv[u]], csem[u], add=True)

    for u in range(2):
        ipf(base_e + u * CHUNK, u)
    for u in range(2):
        do_chunk(base_e + u * CHUNK, u, False, base_e + (u + 2) * CHUNK)

    def body(t2, _):
        for u in range(2):
            t = 2 + 2 * t2 + u
            pf = base_e + (t + 2) * CHUNK
            do_chunk(base_e + t * CHUNK, u, True, pf)
        return 0

    lax.fori_loop(0, (nch - 2) // 2 - 1, body, 0)
    for u in range(2):
        do_chunk(base_e + (nch - 2 + u) * CHUNK, u, True, None)
    sw(0)
    sw(1)

    off = base_e + ((E // 32) // CHUNK) * CHUNK
    pltpu.sync_copy(src_hbm.at[pl.ds(off, 16)], idx_t)
    pltpu.sync_copy(dst_hbm.at[pl.ds(off, 16)], dst_t)
    pltpu.async_copy(x_hbm.at[idx_t], rows_t, sem).wait()
    pltpu.sync_copy(rows_t, acc.at[dst_t], add=True)
    pltpu.sync_copy(ones_v.at[pl.ds(0, 16)], cacc.at[dst_t], add=True)

    plsc.subcore_barrier()
    r0 = s * ROWS_PER_TILE
    pltpu.sync_copy(acc.at[pl.ds(r0, ROWS_PER_TILE)],
                    agg_hbm.at[pl.ds(c * NP + r0, ROWS_PER_TILE)])
    pltpu.sync_copy(cacc.at[pl.ds(r0, ROWS_PER_TILE)],
                    cnt_hbm.at[pl.ds(c * NP + r0, ROWS_PER_TILE)])


_sc_agg0 = pl.kernel(
    _agg0_body,
    out_type=(jax.ShapeDtypeStruct((2 * NP, 128), jnp.float32),
              jax.ShapeDtypeStruct((2 * NP,), jnp.float32)),
    mesh=_mesh,
    scratch_types=[
        pltpu.VMEM((CHUNK,), jnp.int32),
        pltpu.VMEM((CHUNK,), jnp.int32),
        pltpu.VMEM((CHUNK,), jnp.int32),
        pltpu.VMEM((CHUNK,), jnp.int32),
        pltpu.VMEM((CHUNK, 128), jnp.float32),
        pltpu.VMEM((CHUNK, 128), jnp.float32),
        pltpu.VMEM((CHUNK,), jnp.float32),
        pltpu.VMEM((16,), jnp.int32),
        pltpu.VMEM((16,), jnp.int32),
        pltpu.VMEM((16, 128), jnp.float32),
        pltpu.VMEM((16, 128), jnp.float32),
        pltpu.VMEM((ROWS_PER_TILE,), jnp.float32),
        pltpu.VMEM_SHARED((NP, 128), jnp.float32),
        pltpu.VMEM_SHARED((NP,), jnp.float32),
    ] + [pltpu.SemaphoreType.DMA] * 7,
)


def _agg_body(h_hbm, src_hbm, dst_hbm, agg_hbm,
              idx_v, idx_v2, dst_v, dst_v2, rows_v, rows_v2,
              idx_t, dst_t, rows_t,
              zb, acc, sem, ss0, ss1, ix0, ix1):
    c = lax.axis_index("c")
    s = lax.axis_index("s")

    _zero_block(zb)
    _zero_shared(zb, acc, s)
    plsc.subcore_barrier()

    base_e = s * (E // 16)
    rows = [rows_v, rows_v2]
    dstv = [dst_v, dst_v2]
    idxv = [idx_v, idx_v2]
    ssem = [ss0, ss1]
    isem = [ix0, ix1]
    nch = (E // 16) // CHUNK

    def sw(u):
        pltpu.make_async_copy(rows[u], acc.at[dstv[u]], ssem[u]).wait()

    def ipf(t, u):
        pltpu.async_copy(
            src_hbm.at[pl.ds(c * E + base_e + t * CHUNK, CHUNK)],
            idxv[u], isem[u])

    def ipw(u):
        pltpu.make_async_copy(src_hbm.at[pl.ds(0, CHUNK)], idxv[u],
                              isem[u]).wait()

    def do_chunk(t, u, wait_prev, pf_t):
        ipw(u)
        if wait_prev:
            sw(u)
        gather = pltpu.async_copy(h_hbm.at[idxv[u]], rows[u], sem)
        pltpu.sync_copy(dst_hbm.at[pl.ds(base_e + t * CHUNK, CHUNK)],
                        dstv[u])
        gather.wait()
        if pf_t is not None:
            ipf(pf_t, u)
        pltpu.async_copy(rows[u], acc.at[dstv[u]], ssem[u], add=True)

    for u in range(2):
        ipf(u, u)
    for u in range(2):
        do_chunk(u, u, False, u + 2)

    def body(t2, _):
        for u in range(2):
            t = 2 + 2 * t2 + u
            do_chunk(t, u, True, t + 2)
        return 0

    lax.fori_loop(0, (nch - 2) // 2 - 1, body, 0)
    for u in range(2):
        do_chunk(nch - 2 + u, u, True, None)
    sw(0)
    sw(1)

    toff = base_e + ((E // 16) // CHUNK) * CHUNK
    pltpu.sync_copy(src_hbm.at[pl.ds(c * E + toff, 32)], idx_t)
    pltpu.sync_copy(dst_hbm.at[pl.ds(toff, 32)], dst_t)
    pltpu.async_copy(h_hbm.at[idx_t], rows_t, sem).wait()
    pltpu.sync_copy(rows_t, acc.at[dst_t], add=True)

    plsc.subcore_barrier()
    r0 = s * ROWS_PER_TILE
    pltpu.sync_copy(acc.at[pl.ds(r0, ROWS_PER_TILE)],
                    agg_hbm.at[pl.ds(c * NP + r0, ROWS_PER_TILE)])


_sc_agg = pl.kernel(
    _agg_body,
    out_type=jax.ShapeDtypeStruct((2 * NP, 128), jnp.float32),
    mesh=_mesh,
    scratch_types=[
        pltpu.VMEM((CHUNK,), jnp.int32),
        pltpu.VMEM((CHUNK,), jnp.int32),
        pltpu.VMEM((CHUNK,), jnp.int32),
        pltpu.VMEM((CHUNK,), jnp.int32),
        pltpu.VMEM((CHUNK, 128), jnp.float32),
        pltpu.VMEM((CHUNK, 128), jnp.float32),
        pltpu.VMEM((32,), jnp.int32),
        pltpu.VMEM((32,), jnp.int32),
        pltpu.VMEM((32, 128), jnp.float32),
        pltpu.VMEM((16, 128), jnp.float32),
        pltpu.VMEM_SHARED((NP, 128), jnp.float32),
    ] + [pltpu.SemaphoreType.DMA] * 5,
)


ROW_BLK = 2048


def _tc_layer_body(nparts_in, parts_out,
                   agg_ref, cnt_ref, h_ref, wl_ref, wr_ref, b_ref, o_ref):
    cnt = cnt_ref[0] + cnt_ref[1]
    inv = 1.0 / jnp.maximum(cnt, 1.0)
    acc = jnp.zeros((ROW_BLK, 128), jnp.float32)
    for c in range(2):
        acc = acc + jnp.dot(agg_ref[c] * inv[:, None], wl_ref[c],
                            preferred_element_type=jnp.float32)
    for q in range(nparts_in):
        acc = acc + jnp.dot(h_ref[q], wr_ref[q],
                            preferred_element_type=jnp.float32)
    acc = acc + b_ref[0][None, :]
    out = jnp.maximum(acc, 0.0)
    if parts_out:
        o_ref[...] = out[None]
    else:
        o_ref[...] = out


def _tc_layer(agg, cnt, h_parts, wlt, wrt, b, parts_out):
    p_in = h_parts.shape[0]
    grid = (5, 2)
    if parts_out:
        out_shape = jax.ShapeDtypeStruct((2, NP, 128), jnp.float32)
        out_spec = pl.BlockSpec((1, ROW_BLK, 128), lambda i, p: (p, i, 0))
    else:
        out_shape = jax.ShapeDtypeStruct((N, D), jnp.float32)
        out_spec = pl.BlockSpec((ROW_BLK, 128), lambda i, p: (i, p))
    return pl.pallas_call(
        functools.partial(_tc_layer_body, p_in, parts_out),
        grid=grid,
        in_specs=[
            pl.BlockSpec((2, ROW_BLK, 128), lambda i, p: (0, i, 0)),
            pl.BlockSpec((2, ROW_BLK), lambda i, p: (0, i)),
            pl.BlockSpec((p_in, ROW_BLK, 128), lambda i, p: (0, i, 0)),
            pl.BlockSpec((2, 128, 128), lambda i, p: (0, 0, p)),
            pl.BlockSpec((p_in, 128, 128), lambda i, p: (0, 0, p)),
            pl.BlockSpec((1, 128), lambda i, p: (0, p)),
        ],
        out_specs=out_spec,
        out_shape=out_shape,
    )(agg, cnt, h_parts, wlt, wrt, b)


def kernel(x, edge_index, Wl0, Wr0, b0, Wl1, Wr1, b1, Wl2, Wr2, b2):
    src = edge_index[0]
    dst = edge_index[1]
    src_both = jnp.concatenate([src, src + NP])

    agg0, cnt = _sc_agg0(x, src, dst)
    agg0 = agg0.reshape(2, NP, 128)
    cnt = cnt.reshape(2, NP)
    h1 = _tc_layer(agg0, cnt, x.reshape(1, N, 128),
                   jnp.stack([Wl0.T, Wl0.T]), Wr0.T.reshape(1, 128, D),
                   b0.reshape(1, D), parts_out=True)

    agg1 = _sc_agg(h1.reshape(2 * NP, 128), src_both, dst).reshape(2, NP, 128)
    h2 = _tc_layer(agg1, cnt, h1,
                   Wl1.T.reshape(2, 128, D), Wr1.T.reshape(2, 128, D),
                   b1.reshape(1, D), parts_out=True)

    agg2 = _sc_agg(h2.reshape(2 * NP, 128), src_both, dst).reshape(2, NP, 128)
    h3 = _tc_layer(agg2, cnt, h2,
                   Wl2.T.reshape(2, 128, D), Wr2.T.reshape(2, 128, D),
                   b2.reshape(1, D), parts_out=False)

    return h3.reshape(1, N, D)
